# Initial kernel scaffold; baseline (speedup 1.0000x reference)
import jax
import jax.numpy as jnp
from jax.experimental import pallas as pl


def kernel(x, edge_index, edge_weight, ggc_weight, gru_w_ih, gru_w_hh, gru_b_ih, gru_b_hh, lstm_w_ih, lstm_w_hh, lstm_b_ih, lstm_b_hh, lin_w, lin_b):
    raise NotImplementedError("write your pallas kernel here")


# speedup vs baseline: 3.2034x; 3.2034x over previous
import functools

import jax
import jax.numpy as jnp
from jax import lax
from jax.experimental import pallas as pl
from jax.experimental.pallas import tpu as pltpu
from jax.experimental.pallas import tpu_sc as plsc

N = 10000
E = 320000
D = 128
F = 64
O = 32

NC = 2
NS = 16
NW = NC * NS
EPT = E // NW
C = 80
NCHUNK = EPT // C
TROW = 624
ZR = 208


def _sc_body(x_hbm, src_hbm, dst_hbm, w_hbm, summed_hbm, cnt_hbm,
             src_v, dst_v, w_v, rows_v, zrow_v, zcnt_v, ones_v,
             acc_sh, cnt_sh, sem):
    c = lax.axis_index("c")
    s = lax.axis_index("s")
    wid = c * NS + s

    zero16 = jnp.zeros((16,), jnp.float32)
    for i in range(C // 16):
        ones_v[pl.ds(i * 16, 16)] = jnp.ones((16,), jnp.float32)

    def zrow_fill(i, carry):
        for k in range(D // 16):
            zrow_v[i, pl.ds(k * 16, 16)] = zero16
        return carry
    lax.fori_loop(0, ZR, zrow_fill, 0)

    def zcnt_fill(i, carry):
        zcnt_v[pl.ds(i * 16, 16)] = zero16
        return carry
    lax.fori_loop(0, 62, zcnt_fill, 0)
    zcnt_v[pl.ds(984, 16)] = zero16

    for i in range(TROW // ZR):
        pltpu.sync_copy(zrow_v, acc_sh.at[pl.ds(s * TROW + i * ZR, ZR)])

    @pl.when(s == NS - 1)
    def _zero_tail():
        pltpu.sync_copy(zrow_v.at[pl.ds(0, 16)], acc_sh.at[pl.ds(NS * TROW, 16)])

    @pl.when(s < 10)
    def _zero_cnt():
        pltpu.sync_copy(zcnt_v, cnt_sh.at[pl.ds(s * 1000, 1000)])

    plsc.subcore_barrier()

    def chunk_body(i, carry):
        base = wid * EPT + i * C
        pltpu.sync_copy(src_hbm.at[pl.ds(base, C)], src_v)
        pltpu.sync_copy(dst_hbm.at[pl.ds(base, C)], dst_v)
        pltpu.sync_copy(w_hbm.at[pl.ds(base, C)], w_v)
        pltpu.async_copy(x_hbm.at[src_v], rows_v, sem).wait()

        def row_body(j, rcarry):
            wrow = w_v[j]
            for k in range(D // 16):
                rows_v[j, pl.ds(k * 16, 16)] = (
                    rows_v[j, pl.ds(k * 16, 16)] * wrow)
            return rcarry
        lax.fori_loop(0, C, row_body, 0)

        pltpu.sync_copy(rows_v, acc_sh.at[dst_v], add=True)
        pltpu.sync_copy(ones_v, cnt_sh.at[dst_v], add=True)
        return carry
    lax.fori_loop(0, NCHUNK, chunk_body, 0)

    plsc.subcore_barrier()

    for i in range(TROW // ZR):
        pltpu.sync_copy(acc_sh.at[pl.ds(s * TROW + i * ZR, ZR)], zrow_v)
        pltpu.sync_copy(zrow_v, summed_hbm.at[c, pl.ds(s * TROW + i * ZR, ZR)])

    @pl.when(s == NS - 1)
    def _drain_tail():
        pltpu.sync_copy(acc_sh.at[pl.ds(NS * TROW, 16)], zrow_v.at[pl.ds(0, 16)])
        pltpu.sync_copy(zrow_v.at[pl.ds(0, 16)],
                        summed_hbm.at[c, pl.ds(NS * TROW, 16)])

    @pl.when(s < 10)
    def _drain_cnt():
        pltpu.sync_copy(cnt_sh.at[pl.ds(s * 1000, 1000)], zcnt_v)
        pltpu.sync_copy(zcnt_v, cnt_hbm.at[pl.ds(c * N + s * 1000, 1000)])


_sc_agg = pl.kernel(
    _sc_body,
    out_type=[
        jax.ShapeDtypeStruct((NC, N, D), jnp.float32),
        jax.ShapeDtypeStruct((NC * N,), jnp.float32),
    ],
    mesh=plsc.VectorSubcoreMesh(core_axis_name="c", subcore_axis_name="s"),
    scratch_types=[
        pltpu.VMEM((C,), jnp.int32),
        pltpu.VMEM((C,), jnp.int32),
        pltpu.VMEM((C, 16), jnp.float32),
        pltpu.VMEM((C, D), jnp.float32),
        pltpu.VMEM((ZR, D), jnp.float32),
        pltpu.VMEM((1000,), jnp.float32),
        pltpu.VMEM((C,), jnp.float32),
        pltpu.VMEM_SHARED((N, D), jnp.float32),
        pltpu.VMEM_SHARED((N,), jnp.float32),
        pltpu.SemaphoreType.DMA,
    ],
)


BR = 1000


def _tc_body(s2_ref, c2_ref, x_ref, ggc_ref, wih_ref, whh_ref, bih_ref,
             bhh_ref, lwih_ref, lbih_ref, lbhh_ref, lw_ref, lb_ref,
             out_ref, h_ref, c_ref):
    summed = s2_ref[0] + s2_ref[1]
    cnt = c2_ref[0] + c2_ref[1]
    inv = 1.0 / jnp.maximum(cnt, 1.0)
    aggx = summed * inv
    agg = jnp.dot(aggx, ggc_ref[...], preferred_element_type=jnp.float32)
    x = x_ref[...]
    gi = jnp.dot(agg, wih_ref[...], preferred_element_type=jnp.float32) + bih_ref[...]
    gh = jnp.dot(x, whh_ref[...], preferred_element_type=jnp.float32) + bhh_ref[...]
    r = jax.nn.sigmoid(gi[:, :D] + gh[:, :D])
    z = jax.nn.sigmoid(gi[:, D:2 * D] + gh[:, D:2 * D])
    ng = jnp.tanh(gi[:, 2 * D:] + r * gh[:, 2 * D:])
    h_tilde = (1.0 - z) * ng + z * x
    gates = (jnp.dot(h_tilde, lwih_ref[...], preferred_element_type=jnp.float32)
             + lbih_ref[...] + lbhh_ref[...])
    i_g = jax.nn.sigmoid(gates[:, :F])
    g_g = jnp.tanh(gates[:, 2 * F:3 * F])
    o_g = jax.nn.sigmoid(gates[:, 3 * F:])
    c1 = i_g * g_g
    h1 = o_g * jnp.tanh(c1)
    out = (jnp.dot(jnp.maximum(h1, 0.0), lw_ref[...],
                   preferred_element_type=jnp.float32) + lb_ref[...])
    out_ref[...] = out
    h_ref[...] = h1
    c_ref[...] = c1


def _full(shape):
    return pl.BlockSpec(shape, lambda i: tuple(0 for _ in shape))


_tc_update = pl.pallas_call(
    _tc_body,
    grid=(N // BR,),
    in_specs=[
        pl.BlockSpec((NC, BR, D), lambda i: (0, i, 0)),
        pl.BlockSpec((NC, BR, 1), lambda i: (0, i, 0)),
        pl.BlockSpec((BR, D), lambda i: (i, 0)),
        _full((D, D)),
        _full((D, 3 * D)),
        _full((D, 3 * D)),
        _full((1, 3 * D)),
        _full((1, 3 * D)),
        _full((D, 4 * F)),
        _full((1, 4 * F)),
        _full((1, 4 * F)),
        _full((F, O)),
        _full((1, O)),
    ],
    out_specs=[
        pl.BlockSpec((BR, O), lambda i: (i, 0)),
        pl.BlockSpec((BR, F), lambda i: (i, 0)),
        pl.BlockSpec((BR, F), lambda i: (i, 0)),
    ],
    out_shape=[
        jax.ShapeDtypeStruct((N, O), jnp.float32),
        jax.ShapeDtypeStruct((N, F), jnp.float32),
        jax.ShapeDtypeStruct((N, F), jnp.float32),
    ],
)


def kernel(x, edge_index, edge_weight, ggc_weight,
           gru_w_ih, gru_w_hh, gru_b_ih, gru_b_hh,
           lstm_w_ih, lstm_w_hh, lstm_b_ih, lstm_b_hh,
           lin_w, lin_b):
    src = edge_index[0].astype(jnp.int32)
    dst = edge_index[1].astype(jnp.int32)
    w = jnp.broadcast_to(edge_weight.astype(jnp.float32)[:, None], (E, 16))

    summed2, cnt_flat = _sc_agg(x, src, dst, w)
    cnt2 = cnt_flat.reshape(NC, N)

    out, h1, c1 = _tc_update(
        summed2,
        cnt2[..., None],
        x,
        ggc_weight[0],
        gru_w_ih.T,
        gru_w_hh.T,
        gru_b_ih[None, :],
        gru_b_hh[None, :],
        lstm_w_ih.T,
        lstm_b_ih[None, :],
        lstm_b_hh[None, :],
        lin_w.T,
        lin_b[None, :],
    )
    return (out, h1, c1)


# speedup vs baseline: 4.8711x; 1.5206x over previous
import functools

import jax
import jax.numpy as jnp
from jax import lax
from jax.experimental import pallas as pl
from jax.experimental.pallas import tpu as pltpu
from jax.experimental.pallas import tpu_sc as plsc

N = 10000
E = 320000
D = 128
F = 64
O = 32

NC = 2
NS = 16
NW = NC * NS
C = 64
NCHUNK = 156
EPT = NCHUNK * C
XBASE = NW * EPT
TROW = 624


def _sc_body(x_hbm, src_hbm, dst_hbm, w_hbm, summed_hbm, cnt_hbm,
             src_v, dst_v, w_v, rows_v, zcnt_v, ones_v,
             acc_sh, cnt_sh, sem_g, sem_s):
    c = lax.axis_index("c")
    s = lax.axis_index("s")
    wid = c * NS + s

    zero16 = jnp.zeros((16,), jnp.float32)
    for i in range(C // 16):
        ones_v[pl.ds(i * 16, 16)] = jnp.ones((16,), jnp.float32)

    def zfill_row(i, carry):
        for kk in range(D // 16):
            rows_v[0, i, pl.ds(kk * 16, 16)] = zero16
        return carry
    lax.fori_loop(0, C, zfill_row, 0)

    def zcnt_fill(i, carry):
        zcnt_v[pl.ds(i * 16, 16)] = zero16
        return carry
    lax.fori_loop(0, 62, zcnt_fill, 0)
    zcnt_v[pl.ds(984, 16)] = zero16

    for i in range(10):
        nr = C if i < 9 else TROW - 9 * C
        zsrc = rows_v.at[0] if nr == C else rows_v.at[0, pl.ds(0, nr)]
        pltpu.sync_copy(zsrc, acc_sh.at[pl.ds(s * TROW + i * C, nr)])

    @pl.when(s == NS - 1)
    def _zero_tail():
        pltpu.sync_copy(rows_v.at[0, pl.ds(0, 16)],
                        acc_sh.at[pl.ds(NS * TROW, 16)])

    @pl.when(s < 10)
    def _zero_cnt():
        pltpu.sync_copy(zcnt_v, cnt_sh.at[pl.ds(s * 1000, 1000)])

    plsc.subcore_barrier()

    def stage(base, sl):
        base = pl.multiple_of(base, 64)
        wbase = pl.multiple_of(base // 8, 8)
        pltpu.sync_copy(src_hbm.at[pl.ds(base, C)], src_v.at[sl])
        pltpu.sync_copy(dst_hbm.at[pl.ds(base, C)], dst_v.at[sl])
        pltpu.sync_copy(w_hbm.at[pl.ds(wbase, C // 8)], w_v.at[sl])
        pltpu.async_copy(x_hbm.at[src_v.at[sl]], rows_v.at[sl], sem_g.at[sl])

    def wait_gather(sl):
        pltpu.make_async_copy(
            x_hbm.at[src_v.at[sl]], rows_v.at[sl], sem_g.at[sl]).wait()

    def scale(sl):
        def grp_body(g, rcarry):
            for rr in range(8):
                j = g * 8 + rr
                wrow = w_v[sl, g, pl.ds(rr * 16, 16)]
                for kk in range(D // 16):
                    rows_v[sl, j, pl.ds(kk * 16, 16)] = (
                        rows_v[sl, j, pl.ds(kk * 16, 16)] * wrow)
            return rcarry
        lax.fori_loop(0, C // 8, grp_body, 0)

    def scatter(sl):
        pltpu.async_copy(rows_v.at[sl], acc_sh.at[dst_v.at[sl]],
                         sem_s.at[sl], add=True)
        pltpu.async_copy(ones_v, cnt_sh.at[dst_v.at[sl]],
                         sem_s.at[sl], add=True)

    def wait_scatter(sl):
        pltpu.make_async_copy(
            rows_v.at[sl], acc_sh.at[dst_v.at[sl]], sem_s.at[sl]).wait()
        pltpu.make_async_copy(
            ones_v, cnt_sh.at[dst_v.at[sl]], sem_s.at[sl]).wait()

    ebase = wid * EPT
    stage(ebase, 0)
    stage(ebase + C, 1)

    def chunk3_body(t, carry):
        for j in range(3):
            k = 3 * t + j
            b2 = (j + 2) % 3
            wait_gather(j)
            scale(j)
            scatter(j)
            @pl.when(k >= 1)
            def _ws():
                wait_scatter(b2)
            stage(ebase + (k + 2) * C, b2)
        return carry
    lax.fori_loop(0, (NCHUNK - 6) // 3 + 1, chunk3_body, 0)

    wait_scatter(2)
    stage(ebase + (NCHUNK - 1) * C, 2)
    for j in range(3):
        wait_gather(j)
        scale(j)
        scatter(j)
    for sl in range(3):
        wait_scatter(sl)

    @pl.when(wid < 8)
    def _extra():
        stage(XBASE + wid * C, 0)
        wait_gather(0)
        scale(0)
        scatter(0)
        wait_scatter(0)

    plsc.subcore_barrier()

    pend = [None, None, None]
    for i in range(10):
        b = i % 3
        nr = C if i < 9 else TROW - 9 * C
        off = s * TROW + i * C
        if pend[b] is not None:
            pend[b].wait()
        buf = rows_v.at[b] if nr == C else rows_v.at[b, pl.ds(0, nr)]
        pltpu.sync_copy(acc_sh.at[pl.ds(off, nr)], buf)
        pend[b] = pltpu.async_copy(buf, summed_hbm.at[c, pl.ds(off, nr)],
                                   sem_g.at[b])
    for b in range(3):
        if pend[b] is not None:
            pend[b].wait()

    @pl.when(s == NS - 1)
    def _drain_tail():
        pltpu.sync_copy(acc_sh.at[pl.ds(NS * TROW, 16)],
                        rows_v.at[0, pl.ds(0, 16)])
        pltpu.sync_copy(rows_v.at[0, pl.ds(0, 16)],
                        summed_hbm.at[c, pl.ds(NS * TROW, 16)])

    @pl.when(s < 10)
    def _drain_cnt():
        pltpu.sync_copy(cnt_sh.at[pl.ds(s * 1000, 1000)], zcnt_v)
        pltpu.sync_copy(zcnt_v, cnt_hbm.at[pl.ds(c * N + s * 1000, 1000)])


_sc_agg = pl.kernel(
    _sc_body,
    out_type=[
        jax.ShapeDtypeStruct((NC, N, D), jnp.float32),
        jax.ShapeDtypeStruct((NC * N,), jnp.float32),
    ],
    mesh=plsc.VectorSubcoreMesh(core_axis_name="c", subcore_axis_name="s"),
    scratch_types=[
        pltpu.VMEM((3, C), jnp.int32),
        pltpu.VMEM((3, C), jnp.int32),
        pltpu.VMEM((3, C // 8, D), jnp.float32),
        pltpu.VMEM((3, C, D), jnp.float32),
        pltpu.VMEM((1000,), jnp.float32),
        pltpu.VMEM((C,), jnp.float32),
        pltpu.VMEM_SHARED((N, D), jnp.float32),
        pltpu.VMEM_SHARED((N,), jnp.float32),
        pltpu.SemaphoreType.DMA((3,)),
        pltpu.SemaphoreType.DMA((3,)),
    ],
)


BR = 1000


def _tc_body(s2_ref, c2_ref, x_ref, ggc_ref, wih_ref, whh_ref, bih_ref,
             bhh_ref, lwih_ref, lbih_ref, lbhh_ref, lw_ref, lb_ref,
             out_ref, h_ref, c_ref):
    summed = s2_ref[0] + s2_ref[1]
    cnt = c2_ref[0] + c2_ref[1]
    inv = 1.0 / jnp.maximum(cnt, 1.0)
    aggx = summed * inv
    agg = jnp.dot(aggx, ggc_ref[...], preferred_element_type=jnp.float32)
    x = x_ref[...]
    gi = jnp.dot(agg, wih_ref[...], preferred_element_type=jnp.float32) + bih_ref[...]
    gh = jnp.dot(x, whh_ref[...], preferred_element_type=jnp.float32) + bhh_ref[...]
    r = jax.nn.sigmoid(gi[:, :D] + gh[:, :D])
    z = jax.nn.sigmoid(gi[:, D:2 * D] + gh[:, D:2 * D])
    ng = jnp.tanh(gi[:, 2 * D:] + r * gh[:, 2 * D:])
    h_tilde = (1.0 - z) * ng + z * x
    gates = (jnp.dot(h_tilde, lwih_ref[...], preferred_element_type=jnp.float32)
             + lbih_ref[...] + lbhh_ref[...])
    i_g = jax.nn.sigmoid(gates[:, :F])
    g_g = jnp.tanh(gates[:, 2 * F:3 * F])
    o_g = jax.nn.sigmoid(gates[:, 3 * F:])
    c1 = i_g * g_g
    h1 = o_g * jnp.tanh(c1)
    out = (jnp.dot(jnp.maximum(h1, 0.0), lw_ref[...],
                   preferred_element_type=jnp.float32) + lb_ref[...])
    out_ref[...] = out
    h_ref[...] = h1
    c_ref[...] = c1


def _full(shape):
    return pl.BlockSpec(shape, lambda i: tuple(0 for _ in shape))


_tc_update = pl.pallas_call(
    _tc_body,
    grid=(N // BR,),
    in_specs=[
        pl.BlockSpec((NC, BR, D), lambda i: (0, i, 0)),
        pl.BlockSpec((NC, BR, 1), lambda i: (0, i, 0)),
        pl.BlockSpec((BR, D), lambda i: (i, 0)),
        _full((D, D)),
        _full((D, 3 * D)),
        _full((D, 3 * D)),
        _full((1, 3 * D)),
        _full((1, 3 * D)),
        _full((D, 4 * F)),
        _full((1, 4 * F)),
        _full((1, 4 * F)),
        _full((F, O)),
        _full((1, O)),
    ],
    out_specs=[
        pl.BlockSpec((BR, O), lambda i: (i, 0)),
        pl.BlockSpec((BR, F), lambda i: (i, 0)),
        pl.BlockSpec((BR, F), lambda i: (i, 0)),
    ],
    out_shape=[
        jax.ShapeDtypeStruct((N, O), jnp.float32),
        jax.ShapeDtypeStruct((N, F), jnp.float32),
        jax.ShapeDtypeStruct((N, F), jnp.float32),
    ],
)


def kernel(x, edge_index, edge_weight, ggc_weight,
           gru_w_ih, gru_w_hh, gru_b_ih, gru_b_hh,
           lstm_w_ih, lstm_w_hh, lstm_b_ih, lstm_b_hh,
           lin_w, lin_b):
    src = edge_index[0].astype(jnp.int32)
    dst = edge_index[1].astype(jnp.int32)
    w = jnp.broadcast_to(
        edge_weight.astype(jnp.float32)[:, None], (E, 16)).reshape(E // 8, 128)

    summed2, cnt_flat = _sc_agg(x, src, dst, w)
    cnt2 = cnt_flat.reshape(NC, N)

    out, h1, c1 = _tc_update(
        summed2,
        cnt2[..., None],
        x,
        ggc_weight[0],
        gru_w_ih.T,
        gru_w_hh.T,
        gru_b_ih[None, :],
        gru_b_hh[None, :],
        lstm_w_ih.T,
        lstm_b_ih[None, :],
        lstm_b_hh[None, :],
        lin_w.T,
        lin_b[None, :],
    )
    return (out, h1, c1)


# speedup vs baseline: 7.3273x; 1.5042x over previous
import functools

import jax
import jax.numpy as jnp
from jax import lax
from jax.experimental import pallas as pl
from jax.experimental.pallas import tpu as pltpu
from jax.experimental.pallas import tpu_sc as plsc

N = 10000
E = 320000
D = 128
F = 64
O = 32

NC = 2
NS = 16
NW = NC * NS
C = 64
NCHUNK = 156
EPT = NCHUNK * C
XBASE = NW * EPT
TROW = 624


def _sc_body(x_hbm, src_hbm, dst_hbm, w_hbm, summed_hbm, cnt_hbm,
             src_v, dst_v, w_v, rows_v, zcnt_v, ones_v,
             acc_sh, cnt_sh, sem_g, sem_s, sem_i):
    c = lax.axis_index("c")
    s = lax.axis_index("s")
    wid = c * NS + s

    zero16 = jnp.zeros((16,), jnp.float32)
    for i in range(C // 16):
        ones_v[pl.ds(i * 16, 16)] = jnp.ones((16,), jnp.float32)

    def zfill_row(i, carry):
        for kk in range(D // 16):
            rows_v[0, i, pl.ds(kk * 16, 16)] = zero16
        return carry
    lax.fori_loop(0, C, zfill_row, 0)

    def zcnt_fill(i, carry):
        zcnt_v[pl.ds(i * 16, 16)] = zero16
        return carry
    lax.fori_loop(0, 62, zcnt_fill, 0)
    zcnt_v[pl.ds(984, 16)] = zero16

    for i in range(10):
        nr = C if i < 9 else TROW - 9 * C
        zsrc = rows_v.at[0] if nr == C else rows_v.at[0, pl.ds(0, nr)]
        pltpu.sync_copy(zsrc, acc_sh.at[pl.ds(s * TROW + i * C, nr)])

    @pl.when(s == NS - 1)
    def _zero_tail():
        pltpu.sync_copy(rows_v.at[0, pl.ds(0, 16)],
                        acc_sh.at[pl.ds(NS * TROW, 16)])

    @pl.when(s < 10)
    def _zero_cnt():
        pltpu.sync_copy(zcnt_v, cnt_sh.at[pl.ds(s * 1000, 1000)])

    plsc.subcore_barrier()

    def _idx_copies(base, sl):
        base = pl.multiple_of(base, 64)
        wbase = pl.multiple_of(base // 8, 8)
        return (
            (src_hbm.at[pl.ds(base, C)], src_v.at[sl]),
            (dst_hbm.at[pl.ds(base, C)], dst_v.at[sl]),
            (w_hbm.at[pl.ds(wbase, C // 8)], w_v.at[sl]),
        )

    def stage_idx(base, sl):
        for s_ref, d_ref in _idx_copies(base, sl):
            pltpu.async_copy(s_ref, d_ref, sem_i.at[sl])

    def wait_idx(base, sl):
        for s_ref, d_ref in _idx_copies(base, sl):
            pltpu.make_async_copy(s_ref, d_ref, sem_i.at[sl]).wait()

    def gather(sl):
        pltpu.async_copy(x_hbm.at[src_v.at[sl]], rows_v.at[sl], sem_g.at[sl])

    def wait_gather(sl):
        pltpu.make_async_copy(
            x_hbm.at[src_v.at[sl]], rows_v.at[sl], sem_g.at[sl]).wait()

    def scale(sl):
        def grp_body(g, rcarry):
            for rr in range(8):
                j = g * 8 + rr
                wrow = w_v[sl, g, pl.ds(rr * 16, 16)]
                for kk in range(D // 16):
                    rows_v[sl, j, pl.ds(kk * 16, 16)] = (
                        rows_v[sl, j, pl.ds(kk * 16, 16)] * wrow)
            return rcarry
        lax.fori_loop(0, C // 8, grp_body, 0)

    def scatter(sl):
        pltpu.async_copy(rows_v.at[sl], acc_sh.at[dst_v.at[sl]],
                         sem_s.at[sl], add=True)
        pltpu.async_copy(ones_v, cnt_sh.at[dst_v.at[sl]],
                         sem_s.at[sl], add=True)

    def wait_scatter(sl):
        pltpu.make_async_copy(
            rows_v.at[sl], acc_sh.at[dst_v.at[sl]], sem_s.at[sl]).wait()
        pltpu.make_async_copy(
            ones_v, cnt_sh.at[dst_v.at[sl]], sem_s.at[sl]).wait()

    ebase = wid * EPT
    stage_idx(ebase, 0)
    wait_idx(ebase, 0)
    gather(0)
    stage_idx(ebase + C, 1)

    def chunk3_body(t, carry):
        for j in range(3):
            k = 3 * t + j
            j1 = (j + 1) % 3
            j2 = (j + 2) % 3

            @pl.when(k >= 1)
            def _ws():
                wait_scatter(j2)

            @pl.when(k + 2 < NCHUNK)
            def _st():
                stage_idx(ebase + (k + 2) * C, j2)

            @pl.when(k + 1 < NCHUNK)
            def _g():
                wait_idx(ebase + (k + 1) * C, j1)
                gather(j1)

            wait_gather(j)
            scale(j)
            scatter(j)
        return carry
    lax.fori_loop(0, NCHUNK // 3, chunk3_body, 0)

    wait_scatter(2)

    @pl.when(wid < 8)
    def _extra():
        xb = XBASE + wid * C
        stage_idx(xb, 0)
        wait_idx(xb, 0)
        gather(0)
        wait_gather(0)
        scale(0)
        scatter(0)
        wait_scatter(0)

    plsc.subcore_barrier()

    pend = [None, None, None]
    for i in range(10):
        b = i % 3
        nr = C if i < 9 else TROW - 9 * C
        off = s * TROW + i * C
        if pend[b] is not None:
            pend[b].wait()
        buf = rows_v.at[b] if nr == C else rows_v.at[b, pl.ds(0, nr)]
        pltpu.sync_copy(acc_sh.at[pl.ds(off, nr)], buf)
        pend[b] = pltpu.async_copy(buf, summed_hbm.at[c, pl.ds(off, nr)],
                                   sem_g.at[b])
    for b in range(3):
        if pend[b] is not None:
            pend[b].wait()

    @pl.when(s == NS - 1)
    def _drain_tail():
        pltpu.sync_copy(acc_sh.at[pl.ds(NS * TROW, 16)],
                        rows_v.at[0, pl.ds(0, 16)])
        pltpu.sync_copy(rows_v.at[0, pl.ds(0, 16)],
                        summed_hbm.at[c, pl.ds(NS * TROW, 16)])

    @pl.when(s < 10)
    def _drain_cnt():
        pltpu.sync_copy(cnt_sh.at[pl.ds(s * 1000, 1000)], zcnt_v)
        pltpu.sync_copy(zcnt_v, cnt_hbm.at[pl.ds(c * N + s * 1000, 1000)])


_sc_agg = pl.kernel(
    _sc_body,
    out_type=[
        jax.ShapeDtypeStruct((NC, N, D), jnp.float32),
        jax.ShapeDtypeStruct((NC * N,), jnp.float32),
    ],
    mesh=plsc.VectorSubcoreMesh(core_axis_name="c", subcore_axis_name="s"),
    scratch_types=[
        pltpu.VMEM((3, C), jnp.int32),
        pltpu.VMEM((3, C), jnp.int32),
        pltpu.VMEM((3, C // 8, D), jnp.float32),
        pltpu.VMEM((3, C, D), jnp.float32),
        pltpu.VMEM((1000,), jnp.float32),
        pltpu.VMEM((C,), jnp.float32),
        pltpu.VMEM_SHARED((N, D), jnp.float32),
        pltpu.VMEM_SHARED((N,), jnp.float32),
        pltpu.SemaphoreType.DMA((3,)),
        pltpu.SemaphoreType.DMA((3,)),
        pltpu.SemaphoreType.DMA((3,)),
    ],
)


BR = 1000


def _tc_body(s2_ref, c2_ref, x_ref, ggc_ref, wih_ref, whh_ref, bih_ref,
             bhh_ref, lwih_ref, lbih_ref, lbhh_ref, lw_ref, lb_ref,
             out_ref, h_ref, c_ref):
    summed = s2_ref[0] + s2_ref[1]
    cnt = c2_ref[0] + c2_ref[1]
    inv = 1.0 / jnp.maximum(cnt, 1.0)
    aggx = summed * inv
    agg = jnp.dot(aggx, ggc_ref[...], preferred_element_type=jnp.float32)
    x = x_ref[...]
    gi = jnp.dot(agg, wih_ref[...], preferred_element_type=jnp.float32) + bih_ref[...]
    gh = jnp.dot(x, whh_ref[...], preferred_element_type=jnp.float32) + bhh_ref[...]
    r = jax.nn.sigmoid(gi[:, :D] + gh[:, :D])
    z = jax.nn.sigmoid(gi[:, D:2 * D] + gh[:, D:2 * D])
    ng = jnp.tanh(gi[:, 2 * D:] + r * gh[:, 2 * D:])
    h_tilde = (1.0 - z) * ng + z * x
    gates = (jnp.dot(h_tilde, lwih_ref[...], preferred_element_type=jnp.float32)
             + lbih_ref[...] + lbhh_ref[...])
    i_g = jax.nn.sigmoid(gates[:, :F])
    g_g = jnp.tanh(gates[:, 2 * F:3 * F])
    o_g = jax.nn.sigmoid(gates[:, 3 * F:])
    c1 = i_g * g_g
    h1 = o_g * jnp.tanh(c1)
    out = (jnp.dot(jnp.maximum(h1, 0.0), lw_ref[...],
                   preferred_element_type=jnp.float32) + lb_ref[...])
    out_ref[...] = out
    h_ref[...] = h1
    c_ref[...] = c1


def _full(shape):
    return pl.BlockSpec(shape, lambda i: tuple(0 for _ in shape))


_tc_update = pl.pallas_call(
    _tc_body,
    grid=(N // BR,),
    in_specs=[
        pl.BlockSpec((NC, BR, D), lambda i: (0, i, 0)),
        pl.BlockSpec((NC, BR, 1), lambda i: (0, i, 0)),
        pl.BlockSpec((BR, D), lambda i: (i, 0)),
        _full((D, D)),
        _full((D, 3 * D)),
        _full((D, 3 * D)),
        _full((1, 3 * D)),
        _full((1, 3 * D)),
        _full((D, 4 * F)),
        _full((1, 4 * F)),
        _full((1, 4 * F)),
        _full((F, O)),
        _full((1, O)),
    ],
    out_specs=[
        pl.BlockSpec((BR, O), lambda i: (i, 0)),
        pl.BlockSpec((BR, F), lambda i: (i, 0)),
        pl.BlockSpec((BR, F), lambda i: (i, 0)),
    ],
    out_shape=[
        jax.ShapeDtypeStruct((N, O), jnp.float32),
        jax.ShapeDtypeStruct((N, F), jnp.float32),
        jax.ShapeDtypeStruct((N, F), jnp.float32),
    ],
)


def kernel(x, edge_index, edge_weight, ggc_weight,
           gru_w_ih, gru_w_hh, gru_b_ih, gru_b_hh,
           lstm_w_ih, lstm_w_hh, lstm_b_ih, lstm_b_hh,
           lin_w, lin_b):
    src = edge_index[0].astype(jnp.int32)
    dst = edge_index[1].astype(jnp.int32)
    w = jnp.broadcast_to(
        edge_weight.astype(jnp.float32)[:, None], (E, 16)).reshape(E // 8, 128)

    summed2, cnt_flat = _sc_agg(x, src, dst, w)
    cnt2 = cnt_flat.reshape(NC, N)

    out, h1, c1 = _tc_update(
        summed2,
        cnt2[..., None],
        x,
        ggc_weight[0],
        gru_w_ih.T,
        gru_w_hh.T,
        gru_b_ih[None, :],
        gru_b_hh[None, :],
        lstm_w_ih.T,
        lstm_b_ih[None, :],
        lstm_b_hh[None, :],
        lin_w.T,
        lin_b[None, :],
    )
    return (out, h1, c1)


# speedup vs baseline: 10.2727x; 1.4020x over previous
import functools

import jax
import jax.numpy as jnp
from jax import lax
from jax.experimental import pallas as pl
from jax.experimental.pallas import tpu as pltpu
from jax.experimental.pallas import tpu_sc as plsc

N = 10000
E = 320000
D = 128
F = 64
O = 32

NC = 2
NS = 16
NW = NC * NS
C = 64
NCHUNK = 156
EPT = NCHUNK * C
XBASE = NW * EPT
TROW = 624


def _lane_bcast(vec, lane):
    idx = jnp.full((16,), lane, jnp.int32)
    return lax.gather(
        vec, idx[:, None],
        lax.GatherDimensionNumbers(
            offset_dims=(), collapsed_slice_dims=(0,), start_index_map=(0,)),
        slice_sizes=(1,),
        mode=lax.GatherScatterMode.PROMISE_IN_BOUNDS)


def _sc_body(x_hbm, ei_hbm, w_hbm, summed_hbm, cnt_hbm,
             src_v, dst_v, w_v, rows_v, zcnt_v, ones_v,
             acc_sh, cnt_sh, sem_g, sem_s, sem_i):
    c = lax.axis_index("c")
    s = lax.axis_index("s")
    wid = c * NS + s

    zero16 = jnp.zeros((16,), jnp.float32)
    for i in range(C // 16):
        ones_v[pl.ds(i * 16, 16)] = jnp.ones((16,), jnp.float32)

    def zfill_row(i, carry):
        for kk in range(D // 16):
            rows_v[0, i, pl.ds(kk * 16, 16)] = zero16
        return carry
    lax.fori_loop(0, C, zfill_row, 0)

    def zcnt_fill(i, carry):
        zcnt_v[pl.ds(i * 16, 16)] = zero16
        return carry
    lax.fori_loop(0, 62, zcnt_fill, 0)
    zcnt_v[pl.ds(984, 16)] = zero16

    for i in range(10):
        nr = C if i < 9 else TROW - 9 * C
        zsrc = rows_v.at[0] if nr == C else rows_v.at[0, pl.ds(0, nr)]
        pltpu.sync_copy(zsrc, acc_sh.at[pl.ds(s * TROW + i * C, nr)])

    @pl.when(s == NS - 1)
    def _zero_tail():
        pltpu.sync_copy(rows_v.at[0, pl.ds(0, 16)],
                        acc_sh.at[pl.ds(NS * TROW, 16)])

    @pl.when(s < 10)
    def _zero_cnt():
        pltpu.sync_copy(zcnt_v, cnt_sh.at[pl.ds(s * 1000, 1000)])

    plsc.subcore_barrier()

    def _idx_copies(base, sl):
        base = pl.multiple_of(base, 64)
        return (
            (ei_hbm.at[pl.ds(base, C)], src_v.at[sl]),
            (ei_hbm.at[pl.ds(E + base, C)], dst_v.at[sl]),
            (w_hbm.at[pl.ds(base, C)], w_v.at[sl]),
        )

    def stage_idx(base, sl):
        for s_ref, d_ref in _idx_copies(base, sl):
            pltpu.async_copy(s_ref, d_ref, sem_i.at[sl])

    def wait_idx(base, sl):
        for s_ref, d_ref in _idx_copies(base, sl):
            pltpu.make_async_copy(s_ref, d_ref, sem_i.at[sl]).wait()

    def gather(sl):
        pltpu.async_copy(x_hbm.at[src_v.at[sl]], rows_v.at[sl], sem_g.at[sl])

    def wait_gather(sl):
        pltpu.make_async_copy(
            x_hbm.at[src_v.at[sl]], rows_v.at[sl], sem_g.at[sl]).wait()

    def scale(sl):
        def grp_body(g, rcarry):
            wvec = w_v[sl, pl.ds(g * 16, 16)]
            for rr in range(16):
                j = g * 16 + rr
                wrow = _lane_bcast(wvec, rr)
                for kk in range(D // 16):
                    rows_v[sl, j, pl.ds(kk * 16, 16)] = (
                        rows_v[sl, j, pl.ds(kk * 16, 16)] * wrow)
            return rcarry
        lax.fori_loop(0, C // 16, grp_body, 0)

    def scatter(sl):
        pltpu.async_copy(rows_v.at[sl], acc_sh.at[dst_v.at[sl]],
                         sem_s.at[sl], add=True)
        pltpu.async_copy(ones_v, cnt_sh.at[dst_v.at[sl]],
                         sem_s.at[sl], add=True)

    def wait_scatter(sl):
        pltpu.make_async_copy(
            rows_v.at[sl], acc_sh.at[dst_v.at[sl]], sem_s.at[sl]).wait()
        pltpu.make_async_copy(
            ones_v, cnt_sh.at[dst_v.at[sl]], sem_s.at[sl]).wait()

    ebase = wid * EPT
    stage_idx(ebase, 0)
    wait_idx(ebase, 0)
    gather(0)
    stage_idx(ebase + C, 1)

    def chunk3_body(t, carry):
        for j in range(3):
            k = 3 * t + j
            j1 = (j + 1) % 3
            j2 = (j + 2) % 3

            @pl.when(k >= 1)
            def _ws():
                wait_scatter(j2)

            @pl.when(k + 2 < NCHUNK)
            def _st():
                stage_idx(ebase + (k + 2) * C, j2)

            @pl.when(k + 1 < NCHUNK)
            def _g():
                wait_idx(ebase + (k + 1) * C, j1)
                gather(j1)

            wait_gather(j)
            scale(j)
            scatter(j)
        return carry
    lax.fori_loop(0, NCHUNK // 3, chunk3_body, 0)

    wait_scatter(2)

    @pl.when(wid < 8)
    def _extra():
        xb = XBASE + wid * C
        stage_idx(xb, 0)
        wait_idx(xb, 0)
        gather(0)
        wait_gather(0)
        scale(0)
        scatter(0)
        wait_scatter(0)

    plsc.subcore_barrier()

    pend = [None, None, None]
    for i in range(10):
        b = i % 3
        nr = C if i < 9 else TROW - 9 * C
        off = s * TROW + i * C
        if pend[b] is not None:
            pend[b].wait()
        buf = rows_v.at[b] if nr == C else rows_v.at[b, pl.ds(0, nr)]
        pltpu.sync_copy(acc_sh.at[pl.ds(off, nr)], buf)
        pend[b] = pltpu.async_copy(buf, summed_hbm.at[c, pl.ds(off, nr)],
                                   sem_g.at[b])
    for b in range(3):
        if pend[b] is not None:
            pend[b].wait()

    @pl.when(s == NS - 1)
    def _drain_tail():
        pltpu.sync_copy(acc_sh.at[pl.ds(NS * TROW, 16)],
                        rows_v.at[0, pl.ds(0, 16)])
        pltpu.sync_copy(rows_v.at[0, pl.ds(0, 16)],
                        summed_hbm.at[c, pl.ds(NS * TROW, 16)])

    @pl.when(s < 10)
    def _drain_cnt():
        pltpu.sync_copy(cnt_sh.at[pl.ds(s * 1000, 1000)], zcnt_v)
        pltpu.sync_copy(zcnt_v, cnt_hbm.at[pl.ds(c * N + s * 1000, 1000)])


_sc_agg = pl.kernel(
    _sc_body,
    out_type=[
        jax.ShapeDtypeStruct((NC, N, D), jnp.float32),
        jax.ShapeDtypeStruct((NC * N,), jnp.float32),
    ],
    mesh=plsc.VectorSubcoreMesh(core_axis_name="c", subcore_axis_name="s"),
    scratch_types=[
        pltpu.VMEM((3, C), jnp.int32),
        pltpu.VMEM((3, C), jnp.int32),
        pltpu.VMEM((3, C), jnp.float32),
        pltpu.VMEM((3, C, D), jnp.float32),
        pltpu.VMEM((1000,), jnp.float32),
        pltpu.VMEM((C,), jnp.float32),
        pltpu.VMEM_SHARED((N, D), jnp.float32),
        pltpu.VMEM_SHARED((N,), jnp.float32),
        pltpu.SemaphoreType.DMA((3,)),
        pltpu.SemaphoreType.DMA((3,)),
        pltpu.SemaphoreType.DMA((3,)),
    ],
)


BR = 1000


def _tc_body(s2_ref, c2_ref, x_ref, ggc_ref, wih_ref, whh_ref, bih_ref,
             bhh_ref, lwih_ref, lbih_ref, lbhh_ref, lw_ref, lb_ref,
             out_ref, h_ref, c_ref):
    summed = s2_ref[0] + s2_ref[1]
    cnt = c2_ref[0] + c2_ref[1]
    inv = 1.0 / jnp.maximum(cnt, 1.0)
    aggx = summed * inv
    agg = jnp.dot(aggx, ggc_ref[...], preferred_element_type=jnp.float32)
    x = x_ref[...]
    gi = jnp.dot(agg, wih_ref[...], preferred_element_type=jnp.float32) + bih_ref[...]
    gh = jnp.dot(x, whh_ref[...], preferred_element_type=jnp.float32) + bhh_ref[...]
    r = jax.nn.sigmoid(gi[:, :D] + gh[:, :D])
    z = jax.nn.sigmoid(gi[:, D:2 * D] + gh[:, D:2 * D])
    ng = jnp.tanh(gi[:, 2 * D:] + r * gh[:, 2 * D:])
    h_tilde = (1.0 - z) * ng + z * x
    gates = (jnp.dot(h_tilde, lwih_ref[...], preferred_element_type=jnp.float32)
             + lbih_ref[...] + lbhh_ref[...])
    i_g = jax.nn.sigmoid(gates[:, :F])
    g_g = jnp.tanh(gates[:, 2 * F:3 * F])
    o_g = jax.nn.sigmoid(gates[:, 3 * F:])
    c1 = i_g * g_g
    h1 = o_g * jnp.tanh(c1)
    out = (jnp.dot(jnp.maximum(h1, 0.0), lw_ref[...],
                   preferred_element_type=jnp.float32) + lb_ref[...])
    out_ref[...] = out
    h_ref[...] = h1
    c_ref[...] = c1


def _full(shape):
    return pl.BlockSpec(shape, lambda i: tuple(0 for _ in shape))


_tc_update = pl.pallas_call(
    _tc_body,
    grid=(N // BR,),
    in_specs=[
        pl.BlockSpec((NC, BR, D), lambda i: (0, i, 0)),
        pl.BlockSpec((NC, BR, 1), lambda i: (0, i, 0)),
        pl.BlockSpec((BR, D), lambda i: (i, 0)),
        _full((D, D)),
        _full((D, 3 * D)),
        _full((D, 3 * D)),
        _full((1, 3 * D)),
        _full((1, 3 * D)),
        _full((D, 4 * F)),
        _full((1, 4 * F)),
        _full((1, 4 * F)),
        _full((F, O)),
        _full((1, O)),
    ],
    out_specs=[
        pl.BlockSpec((BR, O), lambda i: (i, 0)),
        pl.BlockSpec((BR, F), lambda i: (i, 0)),
        pl.BlockSpec((BR, F), lambda i: (i, 0)),
    ],
    out_shape=[
        jax.ShapeDtypeStruct((N, O), jnp.float32),
        jax.ShapeDtypeStruct((N, F), jnp.float32),
        jax.ShapeDtypeStruct((N, F), jnp.float32),
    ],
)


def kernel(x, edge_index, edge_weight, ggc_weight,
           gru_w_ih, gru_w_hh, gru_b_ih, gru_b_hh,
           lstm_w_ih, lstm_w_hh, lstm_b_ih, lstm_b_hh,
           lin_w, lin_b):
    ei = edge_index.astype(jnp.int32).reshape(2 * E)
    w = edge_weight.astype(jnp.float32)

    summed2, cnt_flat = _sc_agg(x, ei, w)
    cnt2 = cnt_flat.reshape(NC, N)

    out, h1, c1 = _tc_update(
        summed2,
        cnt2[..., None],
        x,
        ggc_weight[0],
        gru_w_ih.T,
        gru_w_hh.T,
        gru_b_ih[None, :],
        gru_b_hh[None, :],
        lstm_w_ih.T,
        lstm_b_ih[None, :],
        lstm_b_hh[None, :],
        lin_w.T,
        lin_b[None, :],
    )
    return (out, h1, c1)


# speedup vs baseline: 10.3905x; 1.0115x over previous
import functools

import jax
import jax.numpy as jnp
from jax import lax
from jax.experimental import pallas as pl
from jax.experimental.pallas import tpu as pltpu
from jax.experimental.pallas import tpu_sc as plsc

N = 10000
E = 320000
D = 128
F = 64
O = 32

NC = 2
NS = 16
NW = NC * NS
C = 64
NCHUNK = 156
EPT = NCHUNK * C
XBASE = NW * EPT
TROW = 624


def _lane_bcast(vec, idx):
    return lax.gather(
        vec, idx[:, None],
        lax.GatherDimensionNumbers(
            offset_dims=(), collapsed_slice_dims=(0,), start_index_map=(0,)),
        slice_sizes=(1,),
        mode=lax.GatherScatterMode.PROMISE_IN_BOUNDS)


def _sc_body(x_hbm, ei_hbm, w_hbm, summed_hbm, cnt_hbm,
             src_v, dst_v, w_v, rows_v, zcnt_v, ones_v,
             acc_sh, cnt_sh, sem_g, sem_s, sem_i):
    c = lax.axis_index("c")
    s = lax.axis_index("s")
    wid = c * NS + s

    zero16 = jnp.zeros((16,), jnp.float32)
    for i in range(C // 16):
        ones_v[pl.ds(i * 16, 16)] = jnp.ones((16,), jnp.float32)

    def zfill_row(i, carry):
        for kk in range(D // 16):
            rows_v[0, i, pl.ds(kk * 16, 16)] = zero16
        return carry
    lax.fori_loop(0, C, zfill_row, 0)

    def zcnt_fill(i, carry):
        zcnt_v[pl.ds(i * 16, 16)] = zero16
        return carry
    lax.fori_loop(0, 62, zcnt_fill, 0)
    zcnt_v[pl.ds(984, 16)] = zero16

    for i in range(10):
        nr = C if i < 9 else TROW - 9 * C
        zsrc = rows_v.at[0] if nr == C else rows_v.at[0, pl.ds(0, nr)]
        pltpu.sync_copy(zsrc, acc_sh.at[pl.ds(s * TROW + i * C, nr)])

    @pl.when(s == NS - 1)
    def _zero_tail():
        pltpu.sync_copy(rows_v.at[0, pl.ds(0, 16)],
                        acc_sh.at[pl.ds(NS * TROW, 16)])

    @pl.when(s < 10)
    def _zero_cnt():
        pltpu.sync_copy(zcnt_v, cnt_sh.at[pl.ds(s * 1000, 1000)])

    plsc.subcore_barrier()

    def _idx_copies(base, sl):
        base = pl.multiple_of(base, 64)
        return (
            (ei_hbm.at[pl.ds(base, C)], src_v.at[sl]),
            (ei_hbm.at[pl.ds(E + base, C)], dst_v.at[sl]),
            (w_hbm.at[pl.ds(base, C)], w_v.at[sl]),
        )

    def stage_idx(base, sl):
        for s_ref, d_ref in _idx_copies(base, sl):
            pltpu.async_copy(s_ref, d_ref, sem_i.at[sl])

    def wait_idx(base, sl):
        for s_ref, d_ref in _idx_copies(base, sl):
            pltpu.make_async_copy(s_ref, d_ref, sem_i.at[sl]).wait()

    def gather(sl):
        pltpu.async_copy(x_hbm.at[src_v.at[sl]], rows_v.at[sl], sem_g.at[sl])

    def wait_gather(sl):
        pltpu.make_async_copy(
            x_hbm.at[src_v.at[sl]], rows_v.at[sl], sem_g.at[sl]).wait()

    def scale(sl):
        wv = tuple(w_v[sl, pl.ds(g * 16, 16)] for g in range(C // 16))

        def row_body(rr, carry):
            idxv = jnp.full((16,), 0, jnp.int32) + rr
            for g in range(C // 16):
                wrow = _lane_bcast(carry[g], idxv)
                j = g * 16 + rr
                for kk in range(D // 16):
                    rows_v[sl, j, pl.ds(kk * 16, 16)] = (
                        rows_v[sl, j, pl.ds(kk * 16, 16)] * wrow)
            return carry
        lax.fori_loop(0, 16, row_body, wv)

    def scatter(sl):
        pltpu.async_copy(rows_v.at[sl], acc_sh.at[dst_v.at[sl]],
                         sem_s.at[sl], add=True)
        pltpu.async_copy(ones_v, cnt_sh.at[dst_v.at[sl]],
                         sem_s.at[sl], add=True)

    def wait_scatter(sl):
        pltpu.make_async_copy(
            rows_v.at[sl], acc_sh.at[dst_v.at[sl]], sem_s.at[sl]).wait()
        pltpu.make_async_copy(
            ones_v, cnt_sh.at[dst_v.at[sl]], sem_s.at[sl]).wait()

    ebase = wid * EPT
    stage_idx(ebase, 0)
    wait_idx(ebase, 0)
    gather(0)
    stage_idx(ebase + C, 1)

    def chunk3_body(t, carry):
        for j in range(3):
            k = 3 * t + j
            j1 = (j + 1) % 3
            j2 = (j + 2) % 3

            @pl.when(k >= 1)
            def _ws():
                wait_scatter(j2)

            @pl.when(k + 2 < NCHUNK)
            def _st():
                stage_idx(ebase + (k + 2) * C, j2)

            @pl.when(k + 1 < NCHUNK)
            def _g():
                wait_idx(ebase + (k + 1) * C, j1)
                gather(j1)

            wait_gather(j)
            scale(j)
            scatter(j)
        return carry
    lax.fori_loop(0, NCHUNK // 3, chunk3_body, 0)

    wait_scatter(2)

    @pl.when(wid < 8)
    def _extra():
        xb = XBASE + wid * C
        stage_idx(xb, 0)
        wait_idx(xb, 0)
        gather(0)
        wait_gather(0)
        scale(0)
        scatter(0)
        wait_scatter(0)

    plsc.subcore_barrier()

    pend = [None, None, None]
    for i in range(10):
        b = i % 3
        nr = C if i < 9 else TROW - 9 * C
        off = s * TROW + i * C
        if pend[b] is not None:
            pend[b].wait()
        buf = rows_v.at[b] if nr == C else rows_v.at[b, pl.ds(0, nr)]
        pltpu.sync_copy(acc_sh.at[pl.ds(off, nr)], buf)
        pend[b] = pltpu.async_copy(buf, summed_hbm.at[c, pl.ds(off, nr)],
                                   sem_g.at[b])
    for b in range(3):
        if pend[b] is not None:
            pend[b].wait()

    @pl.when(s == NS - 1)
    def _drain_tail():
        pltpu.sync_copy(acc_sh.at[pl.ds(NS * TROW, 16)],
                        rows_v.at[0, pl.ds(0, 16)])
        pltpu.sync_copy(rows_v.at[0, pl.ds(0, 16)],
                        summed_hbm.at[c, pl.ds(NS * TROW, 16)])

    @pl.when(s < 10)
    def _drain_cnt():
        pltpu.sync_copy(cnt_sh.at[pl.ds(s * 1000, 1000)], zcnt_v)
        pltpu.sync_copy(zcnt_v, cnt_hbm.at[pl.ds(c * N + s * 1000, 1000)])


_sc_agg = pl.kernel(
    _sc_body,
    out_type=[
        jax.ShapeDtypeStruct((NC, N, D), jnp.float32),
        jax.ShapeDtypeStruct((NC * N,), jnp.float32),
    ],
    mesh=plsc.VectorSubcoreMesh(core_axis_name="c", subcore_axis_name="s"),
    scratch_types=[
        pltpu.VMEM((3, C), jnp.int32),
        pltpu.VMEM((3, C), jnp.int32),
        pltpu.VMEM((3, C), jnp.float32),
        pltpu.VMEM((3, C, D), jnp.float32),
        pltpu.VMEM((1000,), jnp.float32),
        pltpu.VMEM((C,), jnp.float32),
        pltpu.VMEM_SHARED((N, D), jnp.float32),
        pltpu.VMEM_SHARED((N,), jnp.float32),
        pltpu.SemaphoreType.DMA((3,)),
        pltpu.SemaphoreType.DMA((3,)),
        pltpu.SemaphoreType.DMA((3,)),
    ],
)


BR = 1000


def _tc_body(s2_ref, c2_ref, x_ref, ggc_ref, wih_ref, whh_ref, bih_ref,
             bhh_ref, lwih_ref, lbih_ref, lbhh_ref, lw_ref, lb_ref,
             out_ref, h_ref, c_ref):
    summed = s2_ref[0] + s2_ref[1]
    cnt = c2_ref[0] + c2_ref[1]
    inv = 1.0 / jnp.maximum(cnt, 1.0)
    aggx = summed * inv
    agg = jnp.dot(aggx, ggc_ref[...], preferred_element_type=jnp.float32)
    x = x_ref[...]
    gi = jnp.dot(agg, wih_ref[...], preferred_element_type=jnp.float32) + bih_ref[...]
    gh = jnp.dot(x, whh_ref[...], preferred_element_type=jnp.float32) + bhh_ref[...]
    r = jax.nn.sigmoid(gi[:, :D] + gh[:, :D])
    z = jax.nn.sigmoid(gi[:, D:2 * D] + gh[:, D:2 * D])
    ng = jnp.tanh(gi[:, 2 * D:] + r * gh[:, 2 * D:])
    h_tilde = (1.0 - z) * ng + z * x
    gates = (jnp.dot(h_tilde, lwih_ref[...], preferred_element_type=jnp.float32)
             + lbih_ref[...] + lbhh_ref[...])
    i_g = jax.nn.sigmoid(gates[:, :F])
    g_g = jnp.tanh(gates[:, 2 * F:3 * F])
    o_g = jax.nn.sigmoid(gates[:, 3 * F:])
    c1 = i_g * g_g
    h1 = o_g * jnp.tanh(c1)
    out = (jnp.dot(jnp.maximum(h1, 0.0), lw_ref[...],
                   preferred_element_type=jnp.float32) + lb_ref[...])
    out_ref[...] = out
    h_ref[...] = h1
    c_ref[...] = c1


def _full(shape):
    return pl.BlockSpec(shape, lambda i: tuple(0 for _ in shape))


_tc_update = pl.pallas_call(
    _tc_body,
    grid=(N // BR,),
    in_specs=[
        pl.BlockSpec((NC, BR, D), lambda i: (0, i, 0)),
        pl.BlockSpec((NC, BR, 1), lambda i: (0, i, 0)),
        pl.BlockSpec((BR, D), lambda i: (i, 0)),
        _full((D, D)),
        _full((D, 3 * D)),
        _full((D, 3 * D)),
        _full((1, 3 * D)),
        _full((1, 3 * D)),
        _full((D, 4 * F)),
        _full((1, 4 * F)),
        _full((1, 4 * F)),
        _full((F, O)),
        _full((1, O)),
    ],
    out_specs=[
        pl.BlockSpec((BR, O), lambda i: (i, 0)),
        pl.BlockSpec((BR, F), lambda i: (i, 0)),
        pl.BlockSpec((BR, F), lambda i: (i, 0)),
    ],
    out_shape=[
        jax.ShapeDtypeStruct((N, O), jnp.float32),
        jax.ShapeDtypeStruct((N, F), jnp.float32),
        jax.ShapeDtypeStruct((N, F), jnp.float32),
    ],
)


def kernel(x, edge_index, edge_weight, ggc_weight,
           gru_w_ih, gru_w_hh, gru_b_ih, gru_b_hh,
           lstm_w_ih, lstm_w_hh, lstm_b_ih, lstm_b_hh,
           lin_w, lin_b):
    ei = edge_index.astype(jnp.int32).reshape(2 * E)
    w = edge_weight.astype(jnp.float32)

    summed2, cnt_flat = _sc_agg(x, ei, w)
    cnt2 = cnt_flat.reshape(NC, N)

    out, h1, c1 = _tc_update(
        summed2,
        cnt2[..., None],
        x,
        ggc_weight[0],
        gru_w_ih.T,
        gru_w_hh.T,
        gru_b_ih[None, :],
        gru_b_hh[None, :],
        lstm_w_ih.T,
        lstm_b_ih[None, :],
        lstm_b_hh[None, :],
        lin_w.T,
        lin_b[None, :],
    )
    return (out, h1, c1)
